# baseline (speedup 1.0000x reference)
import jax
import jax.numpy as jnp
from jax.experimental import pallas as pl
from jax.experimental.pallas import tpu as pltpu


def _mlp_block(bounds_ref, x_ref, w1_ref, b1_ref, w2_ref, b2_ref,
               w3_ref, b3_ref, w4_ref, b4_ref, out_ref):
    x = x_ref[...]
    h = jnp.dot(x, w1_ref[...], preferred_element_type=jnp.float32)
    h = jnp.maximum(h + b1_ref[...], 0.0)
    h = jnp.dot(h, w2_ref[...], preferred_element_type=jnp.float32)
    h = jnp.maximum(h + b2_ref[...], 0.0)
    h = jnp.dot(h, w3_ref[...], preferred_element_type=jnp.float32)
    h = jnp.maximum(h + b3_ref[...], 0.0)
    v = jnp.dot(h, w4_ref[...], preferred_element_type=jnp.float32)
    v = v + b4_ref[...]
    pts = x[:, :3]
    lo = bounds_ref[0:1, :]
    hi = bounds_ref[1:2, :]
    out = jnp.any((pts < lo) | (pts > hi), axis=-1, keepdims=True)
    out_ref[...] = jnp.where(out, 0.0, v)


def kernel(xt, bounds, W1, b1, W2, b2, W3, b3, W4, b4):
    n, d_in = xt.shape
    d_h = W1.shape[1]
    d_out = W4.shape[1]
    blk = 4096
    grid = n // blk

    full = lambda shape: pl.BlockSpec(shape, lambda i: (0, 0))
    out = pl.pallas_call(
        _mlp_block,
        grid=(grid,),
        in_specs=[
            full((2, 3)),
            pl.BlockSpec((blk, d_in), lambda i: (i, 0)),
            full((d_in, d_h)),
            full((1, d_h)),
            full((d_h, d_h)),
            full((1, d_h)),
            full((d_h, d_h)),
            full((1, d_h)),
            full((d_h, d_out)),
            full((1, d_out)),
        ],
        out_specs=pl.BlockSpec((blk, d_out), lambda i: (i, 0)),
        out_shape=jax.ShapeDtypeStruct((n, d_out), jnp.float32),
    )(bounds, xt, W1, b1.reshape(1, d_h), W2, b2.reshape(1, d_h),
      W3, b3.reshape(1, d_h), W4, b4.reshape(1, d_out))
    return out
